# baseline (speedup 1.0000x reference)
import functools

import jax
import jax.numpy as jnp
from jax import lax
from jax.experimental import pallas as pl
from jax.experimental.pallas import tpu as pltpu
from jax.experimental.pallas import tpu_sc as plsc

B = 1048576
NUM_ACT = 4
NC = 2
NS = 16
L = 16
NW = NC * NS
ROWS = B // NW
CHUNK = 4096
NCHUNK = ROWS // CHUNK

LN2 = 0.6931471805599453
_POLY = (0.03010225, -0.28062917, 1.10479658,
         -2.42079299, 3.49821188, -1.93166642 - 127.0 * LN2)


def _log_f32(p):
    bits = plsc.bitcast(p, jnp.int32)
    ef = (bits >> 23).astype(jnp.float32)
    m = plsc.bitcast((bits & 0x7FFFFF) | 0x3F800000, jnp.float32)
    poly = jnp.float32(_POLY[0])
    for c in _POLY[1:]:
        poly = poly * m + c
    ln = ef * LN2 + poly
    return jnp.where(p == 0.0, -jnp.inf, ln)


def _body(rew_hbm, act_hbm, pro_hbm, out_hbm,
          act_v, rew_v, pro_v, stage_v, red_v, shared, sems):
    c = lax.axis_index("c")
    s = lax.axis_index("s")
    row0 = (c * NS + s) * ROWS
    iota = lax.iota(jnp.int32, L)

    def start(k, buf):
        base = row0 + k * CHUNK
        return (
            pltpu.async_copy(act_hbm.at[pl.ds(base, CHUNK)],
                             act_v.at[buf], sems.at[buf]),
            pltpu.async_copy(rew_hbm.at[pl.ds(base, CHUNK)],
                             rew_v.at[buf], sems.at[buf]),
            pltpu.async_copy(pro_hbm.at[pl.ds(base * 4, CHUNK * 4)],
                             pro_v.at[buf], sems.at[buf]),
        )

    inflight = {0: start(0, 0)}
    acc = jnp.zeros((L,), jnp.float32)
    for k in range(NCHUNK):
        buf = k % 2
        for d in inflight.pop(k):
            d.wait()
        if k + 1 < NCHUNK:
            inflight[k + 1] = start(k + 1, 1 - buf)

        def inner(g, acc, buf=buf):
            a = act_v[buf, pl.ds(g * L, L)]
            r = rew_v[buf, pl.ds(g * L, L)]
            sbase = ((g >> 3) << 9) | ((g & 7) << 4)
            idx = (a << 7) + iota + sbase
            p = plsc.load_gather(pro_v.at[buf], [idx])
            return acc + _log_f32(p) * r

        acc = lax.fori_loop(0, CHUNK // L, inner, acc, unroll=4)

    stage_v[...] = acc * (-1.0 / B)
    pltpu.sync_copy(stage_v, shared.at[s])
    plsc.subcore_barrier()

    @pl.when(s == 0)
    def _():
        pltpu.sync_copy(shared, red_v)
        tot = red_v[0]
        for w in range(1, NS):
            tot = tot + red_v[w]
        stage_v[...] = tot
        pltpu.sync_copy(stage_v, out_hbm.at[pl.ds(c * L, L)])


def _sc_loss_partials(rewards, action, pro_flat):
    mesh = plsc.VectorSubcoreMesh(
        core_axis_name="c", subcore_axis_name="s",
        num_cores=NC, num_subcores=NS)
    return pl.kernel(
        _body,
        out_type=jax.ShapeDtypeStruct((NC * L,), jnp.float32),
        mesh=mesh,
        compiler_params=pltpu.CompilerParams(
            needs_layout_passes=False, use_tc_tiling_on_sc=False),
        scratch_types=[
            pltpu.VMEM((2, CHUNK), jnp.int32),
            pltpu.VMEM((2, CHUNK), jnp.float32),
            pltpu.VMEM((2, CHUNK * NUM_ACT), jnp.float32),
            pltpu.VMEM((L,), jnp.float32),
            pltpu.VMEM((NS, L), jnp.float32),
            pltpu.VMEM_SHARED((NS, L), jnp.float32),
            pltpu.SemaphoreType.DMA((2,)),
        ],
    )(rewards, action, pro_flat)


def kernel(rewards, action, pro):
    pro_perm = jnp.transpose(
        pro.reshape(B // 128, 128, NUM_ACT), (0, 2, 1)).reshape(-1)
    partials = _sc_loss_partials(rewards, action, pro_perm)
    return jnp.sum(partials)


# speedup vs baseline: 1.0713x; 1.0713x over previous
import functools

import jax
import jax.numpy as jnp
from jax import lax
from jax.experimental import pallas as pl
from jax.experimental.pallas import tpu as pltpu
from jax.experimental.pallas import tpu_sc as plsc

B = 1048576
NUM_ACT = 4
NC = 2
NS = 16
L = 16
NW = NC * NS
ROWS = B // NW
CHUNK = 8192
NCHUNK = ROWS // CHUNK

LN2 = 0.6931471805599453
_POLY = (0.03010225, -0.28062917, 1.10479658,
         -2.42079299, 3.49821188, -1.93166642 - 127.0 * LN2)


def _log_f32(p):
    bits = plsc.bitcast(p, jnp.int32)
    ef = (bits >> 23).astype(jnp.float32)
    m = plsc.bitcast((bits & 0x7FFFFF) | 0x3F800000, jnp.float32)
    poly = jnp.float32(_POLY[0])
    for c in _POLY[1:]:
        poly = poly * m + c
    ln = ef * LN2 + poly
    return jnp.where(p == 0.0, -jnp.inf, ln)


def _body(rew_hbm, act_hbm, pro_hbm, out_hbm,
          act_v, rew_v, pro_v, stage_v, red_v, shared, sems):
    c = lax.axis_index("c")
    s = lax.axis_index("s")
    row0 = (c * NS + s) * ROWS
    iota = lax.iota(jnp.int32, L)

    def start(k, buf):
        base = row0 + k * CHUNK
        return (
            pltpu.async_copy(act_hbm.at[pl.ds(base, CHUNK)],
                             act_v.at[buf], sems.at[buf]),
            pltpu.async_copy(rew_hbm.at[pl.ds(base, CHUNK)],
                             rew_v.at[buf], sems.at[buf]),
            pltpu.async_copy(pro_hbm.at[pl.ds(base * 4, CHUNK * 4)],
                             pro_v.at[buf], sems.at[buf]),
        )

    inflight = {0: start(0, 0)}
    acc = jnp.zeros((L,), jnp.float32)
    for k in range(NCHUNK):
        buf = k % 2
        for d in inflight.pop(k):
            d.wait()
        if k + 1 < NCHUNK:
            inflight[k + 1] = start(k + 1, 1 - buf)

        def inner(g, acc, buf=buf):
            a = act_v[buf, pl.ds(g * L, L)]
            r = rew_v[buf, pl.ds(g * L, L)]
            sbase = ((g >> 3) << 9) | ((g & 7) << 4)
            idx = (a << 7) + iota + sbase
            p = plsc.load_gather(pro_v.at[buf], [idx])
            return acc + _log_f32(p) * r

        acc = lax.fori_loop(0, CHUNK // L, inner, acc, unroll=4)

    stage_v[...] = acc * (-1.0 / B)
    pltpu.sync_copy(stage_v, shared.at[s])
    plsc.subcore_barrier()

    @pl.when(s == 0)
    def _():
        pltpu.sync_copy(shared, red_v)
        tot = red_v[0]
        for w in range(1, NS):
            tot = tot + red_v[w]
        stage_v[...] = tot
        pltpu.sync_copy(stage_v, out_hbm.at[pl.ds(c * L, L)])


def _sc_loss_partials(rewards, action, pro_flat):
    mesh = plsc.VectorSubcoreMesh(
        core_axis_name="c", subcore_axis_name="s",
        num_cores=NC, num_subcores=NS)
    return pl.kernel(
        _body,
        out_type=jax.ShapeDtypeStruct((NC * L,), jnp.float32),
        mesh=mesh,
        compiler_params=pltpu.CompilerParams(
            needs_layout_passes=False, use_tc_tiling_on_sc=False),
        scratch_types=[
            pltpu.VMEM((2, CHUNK), jnp.int32),
            pltpu.VMEM((2, CHUNK), jnp.float32),
            pltpu.VMEM((2, CHUNK * NUM_ACT), jnp.float32),
            pltpu.VMEM((L,), jnp.float32),
            pltpu.VMEM((NS, L), jnp.float32),
            pltpu.VMEM_SHARED((NS, L), jnp.float32),
            pltpu.SemaphoreType.DMA((2,)),
        ],
    )(rewards, action, pro_flat)


def kernel(rewards, action, pro):
    pro_perm = jnp.transpose(
        pro.reshape(B // 128, 128, NUM_ACT), (0, 2, 1)).reshape(-1)
    partials = _sc_loss_partials(rewards, action, pro_perm)
    return jnp.sum(partials)


# speedup vs baseline: 1.1016x; 1.0283x over previous
import functools

import jax
import jax.numpy as jnp
from jax import lax
from jax.experimental import pallas as pl
from jax.experimental.pallas import tpu as pltpu
from jax.experimental.pallas import tpu_sc as plsc

B = 1048576
NUM_ACT = 4
NC = 2
NS = 16
L = 16
NW = NC * NS
TC_ROWS = 262144
SC_ROWS = B - TC_ROWS
ROWS = SC_ROWS // NW
CHUNK = 8192
NCHUNK = ROWS // CHUNK
TC_GRID = 8
TC_RB = TC_ROWS // 128 // TC_GRID

LN2 = 0.6931471805599453
_POLY = (0.03010225, -0.28062917, 1.10479658,
         -2.42079299, 3.49821188, -1.93166642 - 127.0 * LN2)


def _log_f32(p):
    bits = plsc.bitcast(p, jnp.int32)
    ef = (bits >> 23).astype(jnp.float32)
    m = plsc.bitcast((bits & 0x7FFFFF) | 0x3F800000, jnp.float32)
    poly = jnp.float32(_POLY[0])
    for c in _POLY[1:]:
        poly = poly * m + c
    ln = ef * LN2 + poly
    return jnp.where(p == 0.0, -jnp.inf, ln)


def _body(rew_hbm, act_hbm, pro_hbm, out_hbm,
          act_v, rew_v, pro_v, stage_v, red_v, shared, sems):
    c = lax.axis_index("c")
    s = lax.axis_index("s")
    row0 = (c * NS + s) * ROWS
    iota = lax.iota(jnp.int32, L)

    def start(k, buf):
        base = row0 + k * CHUNK
        return (
            pltpu.async_copy(act_hbm.at[pl.ds(base, CHUNK)],
                             act_v.at[buf], sems.at[buf]),
            pltpu.async_copy(rew_hbm.at[pl.ds(base, CHUNK)],
                             rew_v.at[buf], sems.at[buf]),
            pltpu.async_copy(pro_hbm.at[pl.ds(base * 4, CHUNK * 4)],
                             pro_v.at[buf], sems.at[buf]),
        )

    inflight = {0: start(0, 0)}
    acc = jnp.zeros((L,), jnp.float32)
    for k in range(NCHUNK):
        buf = k % 2
        for d in inflight.pop(k):
            d.wait()
        if k + 1 < NCHUNK:
            inflight[k + 1] = start(k + 1, 1 - buf)

        def inner(g, acc, buf=buf):
            a = act_v[buf, pl.ds(g * L, L)]
            r = rew_v[buf, pl.ds(g * L, L)]
            sbase = ((g >> 3) << 9) | ((g & 7) << 4)
            idx = (a << 7) + iota + sbase
            p = plsc.load_gather(pro_v.at[buf], [idx])
            return acc + _log_f32(p) * r

        acc = lax.fori_loop(0, CHUNK // L, inner, acc, unroll=4)

    stage_v[...] = acc * (-1.0 / B)
    pltpu.sync_copy(stage_v, shared.at[s])
    plsc.subcore_barrier()

    @pl.when(s == 0)
    def _():
        pltpu.sync_copy(shared, red_v)
        tot = red_v[0]
        for w in range(1, NS):
            tot = tot + red_v[w]
        stage_v[...] = tot
        pltpu.sync_copy(stage_v, out_hbm.at[pl.ds(c * L, L)])


def _sc_loss_partials(rewards, action, pro_flat):
    mesh = plsc.VectorSubcoreMesh(
        core_axis_name="c", subcore_axis_name="s",
        num_cores=NC, num_subcores=NS)
    return pl.kernel(
        _body,
        out_type=jax.ShapeDtypeStruct((NC * L,), jnp.float32),
        mesh=mesh,
        compiler_params=pltpu.CompilerParams(
            needs_layout_passes=False, use_tc_tiling_on_sc=False),
        scratch_types=[
            pltpu.VMEM((2, CHUNK), jnp.int32),
            pltpu.VMEM((2, CHUNK), jnp.float32),
            pltpu.VMEM((2, CHUNK * NUM_ACT), jnp.float32),
            pltpu.VMEM((L,), jnp.float32),
            pltpu.VMEM((NS, L), jnp.float32),
            pltpu.VMEM_SHARED((NS, L), jnp.float32),
            pltpu.SemaphoreType.DMA((2,)),
        ],
    )(rewards, action, pro_flat)


def _tc_body(act_ref, rew_ref, pro_ref, out_ref):
    i = pl.program_id(0)
    a = act_ref[...]
    r = rew_ref[...]
    p = pro_ref[...]
    lp = jnp.log(p)
    arep = jnp.broadcast_to(
        a[:, None, :], (TC_RB, NUM_ACT, 128)).reshape(TC_RB * NUM_ACT, 128)
    rrep = jnp.broadcast_to(
        r[:, None, :], (TC_RB, NUM_ACT, 128)).reshape(TC_RB * NUM_ACT, 128)
    jj = lax.broadcasted_iota(jnp.int32, (TC_RB * NUM_ACT, 128), 0) & 3
    blk = jnp.sum(jnp.where(arep == jj, lp * rrep, 0.0))

    @pl.when(i == 0)
    def _():
        out_ref[0, 0] = blk

    @pl.when(i > 0)
    def _():
        out_ref[0, 0] += blk


def _tc_loss_tail(rewards2, action2, pro_perm2):
    blk0 = SC_ROWS // 128 // TC_RB
    return pl.pallas_call(
        _tc_body,
        grid=(TC_GRID,),
        in_specs=[
            pl.BlockSpec((TC_RB, 128), lambda i: (blk0 + i, 0)),
            pl.BlockSpec((TC_RB, 128), lambda i: (blk0 + i, 0)),
            pl.BlockSpec((TC_RB * NUM_ACT, 128), lambda i: (blk0 + i, 0)),
        ],
        out_specs=pl.BlockSpec((1, 1), lambda i: (0, 0),
                               memory_space=pltpu.SMEM),
        out_shape=jax.ShapeDtypeStruct((1, 1), jnp.float32),
    )(action2, rewards2, pro_perm2)


def kernel(rewards, action, pro):
    pro_perm = jnp.transpose(
        pro.reshape(B // 128, 128, NUM_ACT), (0, 2, 1)).reshape(-1)
    partials = _sc_loss_partials(rewards, action, pro_perm)
    tc_sum = _tc_loss_tail(rewards.reshape(B // 128, 128),
                           action.reshape(B // 128, 128),
                           pro_perm.reshape(B // 128 * NUM_ACT, 128))
    return jnp.sum(partials) - tc_sum[0, 0] * (1.0 / B)


# speedup vs baseline: 1.1707x; 1.0627x over previous
import functools

import jax
import jax.numpy as jnp
from jax import lax
from jax.experimental import pallas as pl
from jax.experimental.pallas import tpu as pltpu
from jax.experimental.pallas import tpu_sc as plsc

B = 1048576
NUM_ACT = 4
NC = 2
NS = 16
L = 16
NW = NC * NS
TC_ROWS = 524288
SC_ROWS = B - TC_ROWS
ROWS = SC_ROWS // NW
CHUNK = 8192
NCHUNK = ROWS // CHUNK
TC_GRID = 8
TC_RB = TC_ROWS // 128 // TC_GRID

LN2 = 0.6931471805599453
_POLY = (0.03010225, -0.28062917, 1.10479658,
         -2.42079299, 3.49821188, -1.93166642 - 127.0 * LN2)


def _log_f32(p):
    bits = plsc.bitcast(p, jnp.int32)
    ef = (bits >> 23).astype(jnp.float32)
    m = plsc.bitcast((bits & 0x7FFFFF) | 0x3F800000, jnp.float32)
    poly = jnp.float32(_POLY[0])
    for c in _POLY[1:]:
        poly = poly * m + c
    ln = ef * LN2 + poly
    return jnp.where(p == 0.0, -jnp.inf, ln)


def _body(rew_hbm, act_hbm, pro_hbm, out_hbm,
          act_v, rew_v, pro_v, stage_v, red_v, shared, sems):
    c = lax.axis_index("c")
    s = lax.axis_index("s")
    row0 = (c * NS + s) * ROWS
    iota = lax.iota(jnp.int32, L)

    def start(k, buf):
        base = row0 + k * CHUNK
        return (
            pltpu.async_copy(act_hbm.at[pl.ds(base, CHUNK)],
                             act_v.at[buf], sems.at[buf]),
            pltpu.async_copy(rew_hbm.at[pl.ds(base, CHUNK)],
                             rew_v.at[buf], sems.at[buf]),
            pltpu.async_copy(pro_hbm.at[pl.ds(base * 4, CHUNK * 4)],
                             pro_v.at[buf], sems.at[buf]),
        )

    inflight = {0: start(0, 0)}
    acc = jnp.zeros((L,), jnp.float32)
    for k in range(NCHUNK):
        buf = k % 2
        for d in inflight.pop(k):
            d.wait()
        if k + 1 < NCHUNK:
            inflight[k + 1] = start(k + 1, 1 - buf)

        def inner(g, acc, buf=buf):
            a = act_v[buf, pl.ds(g * L, L)]
            r = rew_v[buf, pl.ds(g * L, L)]
            sbase = ((g >> 3) << 9) | ((g & 7) << 4)
            idx = (a << 7) + iota + sbase
            p = plsc.load_gather(pro_v.at[buf], [idx])
            return acc + _log_f32(p) * r

        acc = lax.fori_loop(0, CHUNK // L, inner, acc, unroll=4)

    stage_v[...] = acc * (-1.0 / B)
    pltpu.sync_copy(stage_v, shared.at[s])
    plsc.subcore_barrier()

    @pl.when(s == 0)
    def _():
        pltpu.sync_copy(shared, red_v)
        tot = red_v[0]
        for w in range(1, NS):
            tot = tot + red_v[w]
        stage_v[...] = tot
        pltpu.sync_copy(stage_v, out_hbm.at[pl.ds(c * L, L)])


def _sc_loss_partials(rewards, action, pro_flat):
    mesh = plsc.VectorSubcoreMesh(
        core_axis_name="c", subcore_axis_name="s",
        num_cores=NC, num_subcores=NS)
    return pl.kernel(
        _body,
        out_type=jax.ShapeDtypeStruct((NC * L,), jnp.float32),
        mesh=mesh,
        compiler_params=pltpu.CompilerParams(
            needs_layout_passes=False, use_tc_tiling_on_sc=False),
        scratch_types=[
            pltpu.VMEM((2, CHUNK), jnp.int32),
            pltpu.VMEM((2, CHUNK), jnp.float32),
            pltpu.VMEM((2, CHUNK * NUM_ACT), jnp.float32),
            pltpu.VMEM((L,), jnp.float32),
            pltpu.VMEM((NS, L), jnp.float32),
            pltpu.VMEM_SHARED((NS, L), jnp.float32),
            pltpu.SemaphoreType.DMA((2,)),
        ],
    )(rewards, action, pro_flat)


def _tc_body(act_ref, rew_ref, pro_ref, out_ref):
    i = pl.program_id(0)
    a = act_ref[...]
    r = rew_ref[...]
    p4 = pro_ref[...].reshape(TC_RB, NUM_ACT, 128)
    psel = p4[:, 0, :]
    for j in range(1, NUM_ACT):
        psel = jnp.where(a == j, p4[:, j, :], psel)
    blk = jnp.sum(jnp.log(psel) * r)

    @pl.when(i == 0)
    def _():
        out_ref[0, 0] = blk

    @pl.when(i > 0)
    def _():
        out_ref[0, 0] += blk


def _tc_loss_tail(rewards2, action2, pro_perm2):
    blk0 = SC_ROWS // 128 // TC_RB
    return pl.pallas_call(
        _tc_body,
        grid=(TC_GRID,),
        in_specs=[
            pl.BlockSpec((TC_RB, 128), lambda i: (blk0 + i, 0)),
            pl.BlockSpec((TC_RB, 128), lambda i: (blk0 + i, 0)),
            pl.BlockSpec((TC_RB * NUM_ACT, 128), lambda i: (blk0 + i, 0)),
        ],
        out_specs=pl.BlockSpec((1, 1), lambda i: (0, 0),
                               memory_space=pltpu.SMEM),
        out_shape=jax.ShapeDtypeStruct((1, 1), jnp.float32),
    )(action2, rewards2, pro_perm2)


def kernel(rewards, action, pro):
    pro_perm = jnp.transpose(
        pro.reshape(B // 128, 128, NUM_ACT), (0, 2, 1)).reshape(-1)
    partials = _sc_loss_partials(rewards, action, pro_perm)
    tc_sum = _tc_loss_tail(rewards.reshape(B // 128, 128),
                           action.reshape(B // 128, 128),
                           pro_perm.reshape(B // 128 * NUM_ACT, 128))
    return jnp.sum(partials) - tc_sum[0, 0] * (1.0 / B)


# speedup vs baseline: 1.1822x; 1.0099x over previous
import functools

import jax
import jax.numpy as jnp
from jax import lax
from jax.experimental import pallas as pl
from jax.experimental.pallas import tpu as pltpu
from jax.experimental.pallas import tpu_sc as plsc

B = 1048576
NUM_ACT = 4
NC = 2
NS = 16
L = 16
NW = NC * NS
TC_ROWS = 524288
SC_ROWS = B - TC_ROWS
ROWS = SC_ROWS // NW
CHUNK = 8192
NCHUNK = ROWS // CHUNK
TC_GRID = 8
TC_RB = TC_ROWS // 128 // TC_GRID

LN2 = 0.6931471805599453
_POLY = (0.03010225, -0.28062917, 1.10479658,
         -2.42079299, 3.49821188, -1.93166642 - 127.0 * LN2)


def _log_f32(p):
    bits = plsc.bitcast(p, jnp.int32)
    ef = (bits >> 23).astype(jnp.float32)
    m = plsc.bitcast((bits & 0x7FFFFF) | 0x3F800000, jnp.float32)
    poly = jnp.float32(_POLY[0])
    for c in _POLY[1:]:
        poly = poly * m + c
    ln = ef * LN2 + poly
    return jnp.where(p == 0.0, -jnp.inf, ln)


def _body(rew_hbm, act_hbm, pro_hbm, out_hbm,
          act_v, rew_v, pro_v, stage_v, sems):
    c = lax.axis_index("c")
    s = lax.axis_index("s")
    row0 = (c * NS + s) * ROWS
    iota = lax.iota(jnp.int32, L)

    def start(k, buf):
        base = row0 + k * CHUNK
        return (
            pltpu.async_copy(act_hbm.at[pl.ds(base, CHUNK)],
                             act_v.at[buf], sems.at[buf]),
            pltpu.async_copy(rew_hbm.at[pl.ds(base, CHUNK)],
                             rew_v.at[buf], sems.at[buf]),
            pltpu.async_copy(pro_hbm.at[pl.ds(base * 4, CHUNK * 4)],
                             pro_v.at[buf], sems.at[buf]),
        )

    inflight = {0: start(0, 0)}
    acc = jnp.zeros((L,), jnp.float32)
    for k in range(NCHUNK):
        buf = k % 2
        for d in inflight.pop(k):
            d.wait()
        if k + 1 < NCHUNK:
            inflight[k + 1] = start(k + 1, 1 - buf)

        def inner(g, acc, buf=buf):
            a = act_v[buf, pl.ds(g * L, L)]
            r = rew_v[buf, pl.ds(g * L, L)]
            sbase = ((g >> 3) << 9) | ((g & 7) << 4)
            idx = (a << 7) + iota + sbase
            p = plsc.load_gather(pro_v.at[buf], [idx])
            return acc + _log_f32(p) * r

        acc = lax.fori_loop(0, CHUNK // L, inner, acc, unroll=4)

    stage_v[...] = acc * (-1.0 / B)
    pltpu.sync_copy(stage_v, out_hbm.at[pl.ds((c * NS + s) * L, L)])


def _sc_loss_partials(rewards, action, pro_flat):
    mesh = plsc.VectorSubcoreMesh(
        core_axis_name="c", subcore_axis_name="s",
        num_cores=NC, num_subcores=NS)
    return pl.kernel(
        _body,
        out_type=jax.ShapeDtypeStruct((NW * L,), jnp.float32),
        mesh=mesh,
        compiler_params=pltpu.CompilerParams(
            needs_layout_passes=False, use_tc_tiling_on_sc=False),
        scratch_types=[
            pltpu.VMEM((2, CHUNK), jnp.int32),
            pltpu.VMEM((2, CHUNK), jnp.float32),
            pltpu.VMEM((2, CHUNK * NUM_ACT), jnp.float32),
            pltpu.VMEM((L,), jnp.float32),
            pltpu.SemaphoreType.DMA((2,)),
        ],
    )(rewards, action, pro_flat)


def _tc_body(act_ref, rew_ref, pro_ref, out_ref):
    i = pl.program_id(0)
    a = act_ref[...]
    r = rew_ref[...]
    p4 = pro_ref[...].reshape(TC_RB, NUM_ACT, 128)
    psel = p4[:, 0, :]
    for j in range(1, NUM_ACT):
        psel = jnp.where(a == j, p4[:, j, :], psel)
    blk = jnp.sum(jnp.log(psel) * r)

    @pl.when(i == 0)
    def _():
        out_ref[0, 0] = blk

    @pl.when(i > 0)
    def _():
        out_ref[0, 0] += blk


def _tc_loss_tail(rewards2, action2, pro_perm2):
    blk0 = SC_ROWS // 128 // TC_RB
    return pl.pallas_call(
        _tc_body,
        grid=(TC_GRID,),
        in_specs=[
            pl.BlockSpec((TC_RB, 128), lambda i: (blk0 + i, 0)),
            pl.BlockSpec((TC_RB, 128), lambda i: (blk0 + i, 0)),
            pl.BlockSpec((TC_RB * NUM_ACT, 128), lambda i: (blk0 + i, 0)),
        ],
        out_specs=pl.BlockSpec((1, 1), lambda i: (0, 0),
                               memory_space=pltpu.SMEM),
        out_shape=jax.ShapeDtypeStruct((1, 1), jnp.float32),
    )(action2, rewards2, pro_perm2)


def kernel(rewards, action, pro):
    pro_perm = jnp.transpose(
        pro.reshape(B // 128, 128, NUM_ACT), (0, 2, 1)).reshape(-1)
    partials = _sc_loss_partials(rewards, action, pro_perm)
    tc_sum = _tc_loss_tail(rewards.reshape(B // 128, 128),
                           action.reshape(B // 128, 128),
                           pro_perm.reshape(B // 128 * NUM_ACT, 128))
    return jnp.sum(partials) - tc_sum[0, 0] * (1.0 / B)
